# baseline (speedup 1.0000x reference)
import functools

import jax
import jax.numpy as jnp
from jax import lax
from jax.experimental import pallas as pl
from jax.experimental.pallas import tpu as pltpu
from jax.experimental.pallas import tpu_sc as plsc

B = 16
N = 32
K = 4
H = 256
W = 256
HB = H // 2
NC = 2
NS = 16


def _topk_idx_body(scores_ref, out_ref):
    s16 = scores_ref[...]
    s = jnp.broadcast_to(s16[:, None, :], (B, 8, N)).reshape(B * 8, N)
    col = lax.broadcasted_iota(jnp.int32, (B * 8, N), 1)
    picks = []
    for _ in range(K):
        m = jnp.max(s, axis=1, keepdims=True)
        cand = jnp.where(s == m, col, N)
        amin = jnp.min(cand, axis=1, keepdims=True)
        picks.append(amin)
        s = jnp.where(col == amin, -jnp.inf, s)
    row = lax.broadcasted_iota(jnp.int32, (B * 8, HB), 0)
    j = lax.broadcasted_iota(jnp.int32, (B * 8, HB), 1)
    b = row // 8
    h = (row % 8) // 4
    kk = row % 4
    sel = jnp.zeros((B * 8, HB), jnp.int32)
    for k in range(K):
        sel = sel + jnp.where(kk == k, picks[k], 0)
    out_ref[...] = (b * N + sel) * H + h * HB + j


def _topk_idx(iou_scores):
    return pl.pallas_call(
        _topk_idx_body,
        out_shape=jax.ShapeDtypeStruct((B * 8, HB), jnp.int32),
    )(iou_scores)


def _sc_gather_mean(idx, table):
    mesh = plsc.VectorSubcoreMesh(core_axis_name="c", subcore_axis_name="s")

    @functools.partial(
        pl.kernel,
        mesh=mesh,
        out_type=jax.ShapeDtypeStruct((B * H, W), jnp.float32),
        scratch_types=[
            pltpu.VMEM((K, HB), jnp.int32),
            pltpu.VMEM((2, HB, W), jnp.float32),
            pltpu.VMEM((HB, W), jnp.float32),
            pltpu.SemaphoreType.DMA,
            pltpu.SemaphoreType.DMA,
        ],
    )
    def k(idx_hbm, table_hbm, out_hbm, idx_v, stg, acc, sem0, sem1):
        wid = lax.axis_index("s") * NC + lax.axis_index("c")
        b = wid // 2
        h = wid % 2
        pltpu.sync_copy(idx_hbm.at[pl.ds(b * 8 + h * 4, K)], idx_v)
        c0 = pltpu.async_copy(table_hbm.at[idx_v.at[0]], stg.at[0], sem0)
        c1 = pltpu.async_copy(table_hbm.at[idx_v.at[1]], stg.at[1], sem1)
        c0.wait()
        c1.wait()

        def add01(i, _):
            for cc in range(W // 16):
                sl = pl.ds(cc * 16, 16)
                acc[i, sl] = stg[0, i, sl] + stg[1, i, sl]
            return 0

        lax.fori_loop(0, HB, add01, 0)
        c2 = pltpu.async_copy(table_hbm.at[idx_v.at[2]], stg.at[0], sem0)
        c3 = pltpu.async_copy(table_hbm.at[idx_v.at[3]], stg.at[1], sem1)
        c2.wait()
        c3.wait()

        def add23(i, _):
            for cc in range(W // 16):
                sl = pl.ds(cc * 16, 16)
                acc[i, sl] = (acc[i, sl] + (stg[0, i, sl] + stg[1, i, sl])) * 0.25
            return 0

        lax.fori_loop(0, HB, add23, 0)
        pltpu.sync_copy(acc, out_hbm.at[pl.ds(b * H + h * HB, HB)])

    return k(idx, table)


def kernel(iou_scores, mask_preds):
    idx = _topk_idx(iou_scores)
    table = mask_preds.reshape(B * N * H, W)
    out = _sc_gather_mean(idx, table)
    return out.reshape(B, 1, H, W)


# speedup vs baseline: 1.0758x; 1.0758x over previous
import functools

import jax
import jax.numpy as jnp
from jax import lax
from jax.experimental import pallas as pl
from jax.experimental.pallas import tpu as pltpu
from jax.experimental.pallas import tpu_sc as plsc

B = 16
N = 32
K = 4
H = 256
W = 256
HB = H // 2
SR = 32
Q = HB // SR
NC = 2
NS = 16


def _topk_idx_body(scores_ref, out_ref):
    s16 = scores_ref[...]
    R = B * 2 * Q * K
    s = jnp.broadcast_to(s16[:, None, :], (B, R // B, N)).reshape(R, N)
    col = lax.broadcasted_iota(jnp.int32, (R, N), 1)
    picks = []
    for _ in range(K):
        m = jnp.max(s, axis=1, keepdims=True)
        cand = jnp.where(s == m, col, N)
        amin = jnp.min(cand, axis=1, keepdims=True)
        picks.append(amin)
        s = jnp.where(col == amin, -jnp.inf, s)
    row = lax.broadcasted_iota(jnp.int32, (R, SR), 0)
    j = lax.broadcasted_iota(jnp.int32, (R, SR), 1)
    b = row // (2 * Q * K)
    h = (row // (Q * K)) % 2
    q = (row // K) % Q
    kk = row % K
    sel = jnp.zeros((R, SR), jnp.int32)
    for k in range(K):
        sel = sel + jnp.where(kk == k, picks[k], 0)
    out_ref[...] = (b * N + sel) * H + h * HB + q * SR + j


def _topk_idx(iou_scores):
    return pl.pallas_call(
        _topk_idx_body,
        out_shape=jax.ShapeDtypeStruct((B * 2 * Q * K, SR), jnp.int32),
    )(iou_scores)


def _sc_gather_mean(idx, table):
    mesh = plsc.VectorSubcoreMesh(core_axis_name="c", subcore_axis_name="s")

    @functools.partial(
        pl.kernel,
        mesh=mesh,
        out_type=jax.ShapeDtypeStruct((B * H, W), jnp.float32),
        scratch_types=[
            pltpu.VMEM((Q * K, SR), jnp.int32),
            pltpu.VMEM((2, K, SR, W), jnp.float32),
            pltpu.VMEM((2, SR, W), jnp.float32),
            pltpu.SemaphoreType.DMA,
            pltpu.SemaphoreType.DMA,
            pltpu.SemaphoreType.DMA,
            pltpu.SemaphoreType.DMA,
        ],
    )
    def kern(idx_hbm, table_hbm, out_hbm, idx_v, stg, obuf, g0, g1, w0, w1):
        wid = lax.axis_index("s") * NC + lax.axis_index("c")
        b = wid // 2
        h = wid % 2
        gsem = (g0, g1)
        wsem = (w0, w1)
        pltpu.sync_copy(idx_hbm.at[pl.ds((b * 2 + h) * (Q * K), Q * K)], idx_v)

        def gather(q, s):
            return [
                pltpu.async_copy(
                    table_hbm.at[idx_v.at[q * K + k]], stg.at[s, k], gsem[s])
                for k in range(K)
            ]

        gd = {0: gather(0, 0)}
        wb = {}
        for q in range(Q):
            s = q % 2
            for c in gd.pop(q):
                c.wait()
            if q + 1 < Q:
                gd[q + 1] = gather(q + 1, (q + 1) % 2)
            if q - 2 in wb:
                wb.pop(q - 2).wait()

            def body(i, _):
                for cc in range(W // 16):
                    sl = pl.ds(cc * 16, 16)
                    obuf[s, i, sl] = (
                        (stg[s, 0, i, sl] + stg[s, 1, i, sl])
                        + (stg[s, 2, i, sl] + stg[s, 3, i, sl])) * 0.25
                return 0

            lax.fori_loop(0, SR, body, 0)
            dst = out_hbm.at[pl.ds(b * H + h * HB + q * SR, SR)]
            wb[q] = pltpu.async_copy(obuf.at[s], dst, wsem[s])
        for q in (Q - 2, Q - 1):
            wb.pop(q).wait()

    return kern(idx, table)


def kernel(iou_scores, mask_preds):
    idx = _topk_idx(iou_scores)
    table = mask_preds.reshape(B * N * H, W)
    out = _sc_gather_mean(idx, table)
    return out.reshape(B, 1, H, W)
